# baseline (speedup 1.0000x reference)
import functools

import jax
import jax.numpy as jnp
from jax import lax
from jax.experimental import pallas as pl
from jax.experimental.pallas import tpu as pltpu
from jax.experimental.pallas import tpu_sc as plsc

T = 2048
H = 1024
FF = 1024
E = 8
BT = 256
NT = T // BT + E - 1
PAD_T = NT * BT
NW = 32


DUMP = T
YS_ROWS = T + 8


def _router_body(x_ref, wr_ref, xs_ref, eidx_ref):
    x = x_ref[...]
    logits = lax.dot_general(
        x, wr_ref[...], (((1,), (1,)), ((), ())),
        preferred_element_type=jnp.float32,
    )
    topv = jnp.max(logits, axis=1, keepdims=True)
    ids = lax.broadcasted_iota(jnp.int32, logits.shape, 1)
    eidx_ref[...] = jnp.min(jnp.where(logits == topv, ids, E), axis=1,
                            keepdims=True)
    xs_ref[...] = x * jax.nn.sigmoid(topv)


def _swiglu(xb, wg, wu, wd):
    nt = (((1,), (1,)), ((), ()))
    g = lax.dot_general(xb, wg, nt, preferred_element_type=jnp.float32)
    u = lax.dot_general(xb, wu, nt, preferred_element_type=jnp.float32)
    act = (g * jax.nn.sigmoid(g) * u).astype(jnp.bfloat16)
    return lax.dot_general(act, wd, nt, preferred_element_type=jnp.float32)


def _routed_ffn_body(te_ref, xp_ref, wg_ref, wu_ref, wd_ref, out_ref):
    del te_ref
    xb = xp_ref[...].astype(jnp.bfloat16)
    out_ref[...] = _swiglu(xb, wg_ref[0].astype(jnp.bfloat16),
                           wu_ref[0].astype(jnp.bfloat16),
                           wd_ref[0].astype(jnp.bfloat16))


def _shared_ffn_body(x_ref, wg_ref, wu_ref, wd_ref, ysc_ref, out_ref):
    xb = x_ref[...].astype(jnp.bfloat16)
    out_ref[...] = _swiglu(xb, wg_ref[...].astype(jnp.bfloat16),
                           wu_ref[...].astype(jnp.bfloat16),
                           wd_ref[...].astype(jnp.bfloat16)) + ysc_ref[...]


def _sc_worker_id():
    return lax.axis_index("s") * 2 + lax.axis_index("c")


def _make_gather_kernel():
    rows_w = PAD_T // NW
    chunk = 24
    mesh = plsc.VectorSubcoreMesh(core_axis_name="c", subcore_axis_name="s")

    @functools.partial(
        pl.kernel, mesh=mesh,
        out_type=jax.ShapeDtypeStruct((PAD_T, H), jnp.float32),
        scratch_types=[
            pltpu.VMEM((chunk,), jnp.int32),
            pltpu.VMEM((chunk, H), jnp.float32),
            pltpu.SemaphoreType.DMA,
        ],
    )
    def gather_k(xs_hbm, idx_hbm, out_hbm, idx_v, rows_v, sem):
        base = _sc_worker_id() * rows_w
        for c in range(rows_w // chunk):
            off = base + c * chunk
            pltpu.sync_copy(idx_hbm.at[pl.ds(off, chunk)], idx_v)
            pltpu.async_copy(xs_hbm.at[idx_v], rows_v, sem).wait()
            pltpu.sync_copy(rows_v, out_hbm.at[pl.ds(off, chunk)])

    return gather_k


def _make_scatter_kernel():
    rows_w = PAD_T // NW
    chunk = 24
    mesh = plsc.VectorSubcoreMesh(core_axis_name="c", subcore_axis_name="s")

    @functools.partial(
        pl.kernel, mesh=mesh,
        out_type=jax.ShapeDtypeStruct((YS_ROWS, H), jnp.float32),
        scratch_types=[
            pltpu.VMEM((chunk,), jnp.int32),
            pltpu.VMEM((chunk, H), jnp.float32),
            pltpu.SemaphoreType.DMA,
        ],
    )
    def scatter_k(yr_hbm, dst_hbm, out_hbm, idx_v, rows_v, sem):
        base = _sc_worker_id() * rows_w
        for c in range(rows_w // chunk):
            off = base + c * chunk
            pltpu.sync_copy(yr_hbm.at[pl.ds(off, chunk)], rows_v)
            pltpu.sync_copy(dst_hbm.at[pl.ds(off, chunk)], idx_v)
            pltpu.async_copy(rows_v, out_hbm.at[idx_v], sem).wait()

    return scatter_k


def kernel(hidden_states, Wr, Wg, Wu, Wd, Wsg, Wsu, Wsd):
    x = hidden_states

    xs, eidx = pl.pallas_call(
        _router_body,
        out_shape=[
            jax.ShapeDtypeStruct((T, H), jnp.float32),
            jax.ShapeDtypeStruct((T, 1), jnp.int32),
        ],
    )(x, Wr)

    e = eidx[:, 0]
    counts = jnp.bincount(e, length=E).astype(jnp.int32)
    cstart = jnp.concatenate(
        [jnp.zeros((1,), jnp.int32), jnp.cumsum(counts)[:-1].astype(jnp.int32)])
    pcnt = ((counts + BT - 1) // BT) * BT
    pcum = jnp.cumsum(pcnt).astype(jnp.int32)
    pstart = pcum - pcnt
    order = jnp.argsort(e).astype(jnp.int32)
    sorted_e = jnp.sort(e)
    k = jnp.arange(T, dtype=jnp.int32)
    pos_sorted = pstart[sorted_e] + (k - cstart[sorted_e])
    src_full = jnp.zeros((PAD_T,), jnp.int32).at[pos_sorted].set(order)
    dst_full = jnp.full((PAD_T,), DUMP, jnp.int32).at[pos_sorted].set(order)
    te = jnp.clip(
        jnp.searchsorted(pcum, jnp.arange(NT, dtype=jnp.int32) * BT,
                         side="right"),
        0, E - 1).astype(jnp.int32)

    xp = _make_gather_kernel()(xs, src_full)

    grid_spec = pltpu.PrefetchScalarGridSpec(
        num_scalar_prefetch=1,
        grid=(NT,),
        in_specs=[
            pl.BlockSpec((BT, H), lambda i, te_r: (i, 0)),
            pl.BlockSpec((1, FF, H), lambda i, te_r: (te_r[i], 0, 0)),
            pl.BlockSpec((1, FF, H), lambda i, te_r: (te_r[i], 0, 0)),
            pl.BlockSpec((1, H, FF), lambda i, te_r: (te_r[i], 0, 0)),
        ],
        out_specs=pl.BlockSpec((BT, H), lambda i, te_r: (i, 0)),
    )
    yr = pl.pallas_call(
        _routed_ffn_body,
        grid_spec=grid_spec,
        out_shape=jax.ShapeDtypeStruct((PAD_T, H), jnp.float32),
    )(te, xp, Wg, Wu, Wd)

    ysc = _make_scatter_kernel()(yr, dst_full)

    return pl.pallas_call(
        _shared_ffn_body,
        grid=(T // BT,),
        in_specs=[
            pl.BlockSpec((BT, H), lambda i: (i, 0)),
            pl.BlockSpec((FF, H), lambda i: (0, 0)),
            pl.BlockSpec((FF, H), lambda i: (0, 0)),
            pl.BlockSpec((H, FF), lambda i: (0, 0)),
            pl.BlockSpec((BT, H), lambda i: (i, 0)),
        ],
        out_specs=pl.BlockSpec((BT, H), lambda i: (i, 0)),
        out_shape=jax.ShapeDtypeStruct((T, H), jnp.float32),
    )(x, Wsg, Wsu, Wsd, ysc)


# speedup vs baseline: 1.0699x; 1.0699x over previous
import functools

import jax
import jax.numpy as jnp
from jax import lax
from jax.experimental import pallas as pl
from jax.experimental.pallas import tpu as pltpu
from jax.experimental.pallas import tpu_sc as plsc

T = 2048
H = 1024
FF = 1024
E = 8
BT = 256
NT = T // BT + E - 1
PAD_T = NT * BT
NW = 32


DUMP = T
YS_ROWS = T + 8


def _router_body(x_ref, wr_ref, xs_ref, eidx_ref):
    x = x_ref[...]
    logits = lax.dot_general(
        x, wr_ref[...], (((1,), (1,)), ((), ())),
        preferred_element_type=jnp.float32,
    )
    topv = jnp.max(logits, axis=1, keepdims=True)
    ids = lax.broadcasted_iota(jnp.int32, logits.shape, 1)
    eidx_ref[...] = jnp.min(jnp.where(logits == topv, ids, E), axis=1,
                            keepdims=True)
    xs_ref[...] = x * jax.nn.sigmoid(topv)


def _swiglu(xb, wg, wu, wd):
    nt = (((1,), (1,)), ((), ()))
    g = lax.dot_general(xb, wg, nt, preferred_element_type=jnp.float32)
    u = lax.dot_general(xb, wu, nt, preferred_element_type=jnp.float32)
    act = (g * jax.nn.sigmoid(g) * u).astype(jnp.bfloat16)
    return lax.dot_general(act, wd, nt, preferred_element_type=jnp.float32)


def _routed_ffn_body(te_ref, xp_ref, wg_ref, wu_ref, wd_ref, out_ref):
    del te_ref
    xb = xp_ref[...].astype(jnp.bfloat16)
    out_ref[...] = _swiglu(xb, wg_ref[0].astype(jnp.bfloat16),
                           wu_ref[0].astype(jnp.bfloat16),
                           wd_ref[0].astype(jnp.bfloat16))


def _shared_ffn_body(x_ref, wg_ref, wu_ref, wd_ref, out_ref):
    xb = x_ref[...].astype(jnp.bfloat16)
    out_ref[...] = _swiglu(xb, wg_ref[...].astype(jnp.bfloat16),
                           wu_ref[...].astype(jnp.bfloat16),
                           wd_ref[...].astype(jnp.bfloat16))


def _sc_worker_id():
    return lax.axis_index("s") * 2 + lax.axis_index("c")


def _make_gather_kernel():
    rows_w = PAD_T // NW
    mesh = plsc.VectorSubcoreMesh(core_axis_name="c", subcore_axis_name="s")

    @functools.partial(
        pl.kernel, mesh=mesh,
        out_type=jax.ShapeDtypeStruct((PAD_T, H), jnp.float32),
        scratch_types=[
            pltpu.VMEM((rows_w,), jnp.int32),
            pltpu.VMEM((rows_w, H), jnp.float32),
            pltpu.SemaphoreType.DMA,
        ],
    )
    def gather_k(xs_hbm, idx_hbm, out_hbm, idx_v, rows_v, sem):
        off = _sc_worker_id() * rows_w
        pltpu.sync_copy(idx_hbm.at[pl.ds(off, rows_w)], idx_v)
        pltpu.async_copy(xs_hbm.at[idx_v], rows_v, sem).wait()
        pltpu.sync_copy(rows_v, out_hbm.at[pl.ds(off, rows_w)])

    return gather_k


def _make_scatter_kernel():
    rows_w = PAD_T // NW
    mesh = plsc.VectorSubcoreMesh(core_axis_name="c", subcore_axis_name="s")

    @functools.partial(
        pl.kernel, mesh=mesh,
        out_type=jax.ShapeDtypeStruct((YS_ROWS, H), jnp.float32),
        scratch_types=[
            pltpu.VMEM((rows_w,), jnp.int32),
            pltpu.VMEM((rows_w, H), jnp.float32),
            pltpu.SemaphoreType.DMA,
        ],
    )
    def scatter_k(yr_hbm, dst_hbm, out_hbm, idx_v, rows_v, sem):
        off = _sc_worker_id() * rows_w
        pltpu.sync_copy(yr_hbm.at[pl.ds(off, rows_w)], rows_v)
        pltpu.sync_copy(dst_hbm.at[pl.ds(off, rows_w)], idx_v)
        pltpu.async_copy(rows_v, out_hbm.at[idx_v], sem).wait()

    return scatter_k


def _add_body(a_ref, b_ref, o_ref):
    o_ref[...] = a_ref[...] + b_ref[...]


def kernel(hidden_states, Wr, Wg, Wu, Wd, Wsg, Wsu, Wsd):
    x = hidden_states

    xs, eidx = pl.pallas_call(
        _router_body,
        out_shape=[
            jax.ShapeDtypeStruct((T, H), jnp.float32),
            jax.ShapeDtypeStruct((T, 1), jnp.int32),
        ],
    )(x, Wr)

    e = eidx[:, 0]
    counts = jnp.bincount(e, length=E).astype(jnp.int32)
    cstart = jnp.concatenate(
        [jnp.zeros((1,), jnp.int32), jnp.cumsum(counts)[:-1].astype(jnp.int32)])
    pcnt = ((counts + BT - 1) // BT) * BT
    pcum = jnp.cumsum(pcnt).astype(jnp.int32)
    pstart = pcum - pcnt
    order = jnp.argsort(e).astype(jnp.int32)
    sorted_e = jnp.sort(e)
    k = jnp.arange(T, dtype=jnp.int32)
    pos_sorted = pstart[sorted_e] + (k - cstart[sorted_e])
    src_full = jnp.zeros((PAD_T,), jnp.int32).at[pos_sorted].set(order)
    dst_full = jnp.full((PAD_T,), DUMP, jnp.int32).at[pos_sorted].set(order)
    te = jnp.clip(
        jnp.searchsorted(pcum, jnp.arange(NT, dtype=jnp.int32) * BT,
                         side="right"),
        0, E - 1).astype(jnp.int32)

    xp = _make_gather_kernel()(xs, src_full)

    ysh = pl.pallas_call(
        _shared_ffn_body,
        grid=(T // BT,),
        in_specs=[
            pl.BlockSpec((BT, H), lambda i: (i, 0)),
            pl.BlockSpec((FF, H), lambda i: (0, 0)),
            pl.BlockSpec((FF, H), lambda i: (0, 0)),
            pl.BlockSpec((H, FF), lambda i: (0, 0)),
        ],
        out_specs=pl.BlockSpec((BT, H), lambda i: (i, 0)),
        out_shape=jax.ShapeDtypeStruct((T, H), jnp.float32),
    )(x, Wsg, Wsu, Wsd)

    grid_spec = pltpu.PrefetchScalarGridSpec(
        num_scalar_prefetch=1,
        grid=(NT,),
        in_specs=[
            pl.BlockSpec((BT, H), lambda i, te_r: (i, 0)),
            pl.BlockSpec((1, FF, H), lambda i, te_r: (te_r[i], 0, 0)),
            pl.BlockSpec((1, FF, H), lambda i, te_r: (te_r[i], 0, 0)),
            pl.BlockSpec((1, H, FF), lambda i, te_r: (te_r[i], 0, 0)),
        ],
        out_specs=pl.BlockSpec((BT, H), lambda i, te_r: (i, 0)),
    )
    yr = pl.pallas_call(
        _routed_ffn_body,
        grid_spec=grid_spec,
        out_shape=jax.ShapeDtypeStruct((PAD_T, H), jnp.float32),
    )(te, xp, Wg, Wu, Wd)

    ysc = _make_scatter_kernel()(yr, dst_full)

    return pl.pallas_call(
        _add_body,
        grid=(T // BT,),
        in_specs=[
            pl.BlockSpec((BT, H), lambda i: (i, 0)),
            pl.BlockSpec((BT, H), lambda i: (i, 0)),
        ],
        out_specs=pl.BlockSpec((BT, H), lambda i: (i, 0)),
        out_shape=jax.ShapeDtypeStruct((T, H), jnp.float32),
    )(ysh, ysc)
